# Initial kernel scaffold; baseline (speedup 1.0000x reference)
import jax
import jax.numpy as jnp
from jax.experimental import pallas as pl


def kernel(x, edge_index, W1, b1, W2, b2):
    raise NotImplementedError("write your pallas kernel here")


# speedup vs baseline: 21.9041x; 21.9041x over previous
import functools

import jax
import jax.numpy as jnp
from jax import lax
from jax.experimental import pallas as pl
from jax.experimental.pallas import tpu as pltpu
from jax.experimental.pallas import tpu_sc as plsc

N = 10000
D = 128
E = 320000
NP = 10240
EP = 327680
NC = 2
NS = 16
NW = NC * NS
C = 128
K = EP // (NW * C)
RT = NP // NS

_mesh = plsc.VectorSubcoreMesh(core_axis_name="c", subcore_axis_name="s")


@functools.partial(
    pl.kernel,
    out_type=jax.ShapeDtypeStruct((NC, NP), jnp.float32),
    mesh=_mesh,
    scratch_types=[
        pltpu.VMEM((K, C), jnp.int32),
        pltpu.VMEM((C,), jnp.float32),
        pltpu.VMEM_SHARED((NP,), jnp.float32),
    ],
)
def _sc_counts(dst_hbm, zeros1_hbm, cnt_hbm, idx_v, ones_v, cnt_sh):
    cid = lax.axis_index("c")
    sid = lax.axis_index("s")
    wid = cid * NS + sid
    pltpu.sync_copy(zeros1_hbm.at[pl.ds(sid * RT, RT)],
                    cnt_sh.at[pl.ds(sid * RT, RT)])
    for j in range(C // 16):
        ones_v[pl.ds(j * 16, 16)] = jnp.ones((16,), jnp.float32)
    pltpu.sync_copy(dst_hbm.at[wid], idx_v)
    plsc.subcore_barrier()

    def body(g, carry):
        pltpu.sync_copy(ones_v, cnt_sh.at[idx_v.at[g]], add=True)
        return carry

    lax.fori_loop(0, K, body, 0)
    plsc.subcore_barrier()
    pltpu.sync_copy(cnt_sh.at[pl.ds(sid * RT, RT)],
                    cnt_hbm.at[cid, pl.ds(sid * RT, RT)])


@functools.partial(
    pl.kernel,
    out_type=jax.ShapeDtypeStruct((NC, NP, D), jnp.float32),
    mesh=_mesh,
    scratch_types=[
        pltpu.VMEM((K, C), jnp.int32),
        pltpu.VMEM((K, C), jnp.int32),
        pltpu.VMEM((C, D), jnp.float32),
        pltpu.VMEM_SHARED((NP, D), jnp.float32),
        pltpu.SemaphoreType.DMA,
    ],
)
def _sc_agg(table_hbm, src_hbm, dst_hbm, zeros2_hbm, out_hbm,
            src_v, dst_v, rows_v, acc_sh, sem):
    cid = lax.axis_index("c")
    sid = lax.axis_index("s")
    wid = cid * NS + sid
    pltpu.sync_copy(zeros2_hbm.at[pl.ds(sid * RT, RT)],
                    acc_sh.at[pl.ds(sid * RT, RT)])
    pltpu.sync_copy(src_hbm.at[wid], src_v)
    pltpu.sync_copy(dst_hbm.at[wid], dst_v)
    plsc.subcore_barrier()

    def body(g, carry):
        pltpu.async_copy(table_hbm.at[src_v.at[g]], rows_v, sem).wait()
        pltpu.sync_copy(rows_v, acc_sh.at[dst_v.at[g]], add=True)
        return carry

    lax.fori_loop(0, K, body, 0)
    plsc.subcore_barrier()
    pltpu.sync_copy(acc_sh.at[pl.ds(sid * RT, RT)],
                    out_hbm.at[cid, pl.ds(sid * RT, RT)])


BLK = 512
_GRID = NP // BLK


def _dis_of(cnt_ref):
    deg = 1.0 + cnt_ref[0, :] + cnt_ref[1, :]
    return lax.rsqrt(deg)


def _tc_s1_body(cnt_ref, x_ref, w_ref, out_ref):
    dis = _dis_of(cnt_ref)
    xw = jnp.dot(x_ref[...], w_ref[...], preferred_element_type=jnp.float32)
    out_ref[...] = xw * dis[:, None]


def _tc_mid_body(cnt_ref, agg_ref, s_ref, b_ref, w_ref, out_ref):
    dis = _dis_of(cnt_ref)
    tot = agg_ref[0] + agg_ref[1] + s_ref[...]
    h = jnp.maximum(dis[:, None] * tot + b_ref[...], 0.0)
    hw = jnp.dot(h, w_ref[...], preferred_element_type=jnp.float32)
    out_ref[...] = hw * dis[:, None]


def _tc_out_body(cnt_ref, agg_ref, s_ref, b_ref, out_ref):
    dis = _dis_of(cnt_ref)
    o = dis[:, None] * (agg_ref[0] + agg_ref[1] + s_ref[...]) + b_ref[...]
    m = jnp.max(o, axis=1, keepdims=True)
    ex = jnp.exp(o - m)
    out_ref[...] = ex / jnp.sum(ex, axis=1, keepdims=True)


_cnt_spec = pl.BlockSpec((NC, BLK), lambda i: (0, i))
_row_spec = pl.BlockSpec((BLK, D), lambda i: (i, 0))
_agg_spec = pl.BlockSpec((NC, BLK, D), lambda i: (0, i, 0))
_w_spec = pl.BlockSpec((D, D), lambda i: (0, 0))
_b_spec = pl.BlockSpec((1, D), lambda i: (0, 0))
_out_sds = jax.ShapeDtypeStruct((NP, D), jnp.float32)

_tc_s1 = pl.pallas_call(
    _tc_s1_body, grid=(_GRID,),
    in_specs=[_cnt_spec, _row_spec, _w_spec],
    out_specs=_row_spec, out_shape=_out_sds)

_tc_mid = pl.pallas_call(
    _tc_mid_body, grid=(_GRID,),
    in_specs=[_cnt_spec, _agg_spec, _row_spec, _b_spec, _w_spec],
    out_specs=_row_spec, out_shape=_out_sds)

_tc_out = pl.pallas_call(
    _tc_out_body, grid=(_GRID,),
    in_specs=[_cnt_spec, _agg_spec, _row_spec, _b_spec],
    out_specs=_row_spec, out_shape=_out_sds)


def kernel(x, edge_index, W1, b1, W2, b2):
    src = edge_index[0].astype(jnp.int32)
    dst = edge_index[1].astype(jnp.int32)
    npad = EP - E
    pad_i = jnp.arange(npad, dtype=jnp.int32)
    src_p = jnp.concatenate([src, pad_i % N]).reshape(NW, K, C)
    dst_p = jnp.concatenate([dst, N + pad_i % (NP - N)]).reshape(NW, K, C)

    zeros1 = jnp.zeros((NP,), jnp.float32)
    zeros2 = jnp.zeros((NP, D), jnp.float32)
    xp = jnp.concatenate([x, jnp.zeros((NP - N, D), jnp.float32)], axis=0)

    cnt = _sc_counts(dst_p, zeros1)

    s1 = _tc_s1(cnt, xp, W1)
    agg1 = _sc_agg(s1, src_p, dst_p, zeros2)
    s2 = _tc_mid(cnt, agg1, s1, b1.reshape(1, D), W2)
    agg2 = _sc_agg(s2, src_p, dst_p, zeros2)
    out = _tc_out(cnt, agg2, s2, b2.reshape(1, D))
    return out[:N]


# speedup vs baseline: 29.3618x; 1.3405x over previous
import functools

import jax
import jax.numpy as jnp
from jax import lax
from jax.experimental import pallas as pl
from jax.experimental.pallas import tpu as pltpu
from jax.experimental.pallas import tpu_sc as plsc

N = 10000
D = 128
E = 320000
NP = 10240
EP = 327680
NC = 2
NS = 16
NW = NC * NS
C = 128
K = EP // (NW * C)
KS = 16
RT = NP // NS

_mesh = plsc.VectorSubcoreMesh(core_axis_name="c", subcore_axis_name="s")


@functools.partial(
    pl.kernel,
    out_type=jax.ShapeDtypeStruct((NC, NP), jnp.float32),
    mesh=_mesh,
    scratch_types=[
        pltpu.VMEM((K, C), jnp.int32),
        pltpu.VMEM((C,), jnp.float32),
        pltpu.VMEM_SHARED((NP,), jnp.float32),
    ],
)
def _sc_counts(dst_hbm, zeros1_hbm, cnt_hbm, idx_v, ones_v, cnt_sh):
    cid = lax.axis_index("c")
    sid = lax.axis_index("s")
    wid = cid * NS + sid
    pltpu.sync_copy(zeros1_hbm.at[pl.ds(sid * RT, RT)],
                    cnt_sh.at[pl.ds(sid * RT, RT)])
    for j in range(C // 16):
        ones_v[pl.ds(j * 16, 16)] = jnp.ones((16,), jnp.float32)
    pltpu.sync_copy(dst_hbm.at[wid], idx_v)
    plsc.subcore_barrier()

    def body(g, carry):
        pltpu.sync_copy(ones_v, cnt_sh.at[idx_v.at[g]], add=True)
        return carry

    lax.fori_loop(0, K, body, 0)
    plsc.subcore_barrier()
    pltpu.sync_copy(cnt_sh.at[pl.ds(sid * RT, RT)],
                    cnt_hbm.at[cid, pl.ds(sid * RT, RT)])


@functools.partial(
    pl.kernel,
    out_type=jax.ShapeDtypeStruct((NC, NP, D), jnp.float32),
    mesh=_mesh,
    scratch_types=[
        pltpu.VMEM((KS, C), jnp.int32),
        pltpu.VMEM((KS, C), jnp.int32),
        pltpu.VMEM((C, D), jnp.float32),
        pltpu.VMEM((C, D), jnp.float32),
        pltpu.VMEM_SHARED((NP, D), jnp.float32),
        pltpu.SemaphoreType.DMA,
        pltpu.SemaphoreType.DMA,
    ],
)
def _sc_agg(table_hbm, src_hbm, dst_hbm, zeros2_hbm, out_hbm,
            src_v, dst_v, rows0_v, rows1_v, acc_sh, sem0, sem1):
    cid = lax.axis_index("c")
    sid = lax.axis_index("s")
    wid = cid * NS + sid
    rows = (rows0_v, rows1_v)
    sems = (sem0, sem1)
    pltpu.sync_copy(zeros2_hbm.at[pl.ds(sid * RT, RT)],
                    acc_sh.at[pl.ds(sid * RT, RT)])
    plsc.subcore_barrier()

    def stage(s, carry):
        pltpu.sync_copy(src_hbm.at[wid, pl.ds(s * KS, KS)], src_v)
        pltpu.sync_copy(dst_hbm.at[wid, pl.ds(s * KS, KS)], dst_v)
        pltpu.async_copy(table_hbm.at[src_v.at[0]], rows0_v, sem0)
        pltpu.async_copy(table_hbm.at[src_v.at[1]], rows1_v, sem1)

        def body(i, carry2):
            g = i * 2
            for b in (0, 1):
                gb = g + b
                pltpu.make_async_copy(table_hbm.at[src_v.at[gb]],
                                      rows[b], sems[b]).wait()
                pltpu.sync_copy(rows[b], acc_sh.at[dst_v.at[gb]], add=True)

                @pl.when(gb + 2 < KS)
                def _():
                    pltpu.async_copy(table_hbm.at[src_v.at[gb + 2]],
                                     rows[b], sems[b])
            return carry2

        lax.fori_loop(0, KS // 2, body, 0)
        return carry

    lax.fori_loop(0, K // KS, stage, 0)
    plsc.subcore_barrier()
    pltpu.sync_copy(acc_sh.at[pl.ds(sid * RT, RT)],
                    out_hbm.at[cid, pl.ds(sid * RT, RT)])


BLK = 512
_GRID = NP // BLK


def _dis_of(cnt_ref):
    deg = 1.0 + cnt_ref[0, :] + cnt_ref[1, :]
    return lax.rsqrt(deg)


def _tc_s1_body(cnt_ref, x_ref, w_ref, out_ref):
    dis = _dis_of(cnt_ref)
    xw = jnp.dot(x_ref[...], w_ref[...], preferred_element_type=jnp.float32)
    out_ref[...] = xw * dis[:, None]


def _tc_mid_body(cnt_ref, agg_ref, s_ref, b_ref, w_ref, out_ref):
    dis = _dis_of(cnt_ref)
    tot = agg_ref[0] + agg_ref[1] + s_ref[...]
    h = jnp.maximum(dis[:, None] * tot + b_ref[...], 0.0)
    hw = jnp.dot(h, w_ref[...], preferred_element_type=jnp.float32)
    out_ref[...] = hw * dis[:, None]


def _tc_out_body(cnt_ref, agg_ref, s_ref, b_ref, out_ref):
    dis = _dis_of(cnt_ref)
    o = dis[:, None] * (agg_ref[0] + agg_ref[1] + s_ref[...]) + b_ref[...]
    m = jnp.max(o, axis=1, keepdims=True)
    ex = jnp.exp(o - m)
    out_ref[...] = ex / jnp.sum(ex, axis=1, keepdims=True)


_cnt_spec = pl.BlockSpec((NC, BLK), lambda i: (0, i))
_row_spec = pl.BlockSpec((BLK, D), lambda i: (i, 0))
_agg_spec = pl.BlockSpec((NC, BLK, D), lambda i: (0, i, 0))
_w_spec = pl.BlockSpec((D, D), lambda i: (0, 0))
_b_spec = pl.BlockSpec((1, D), lambda i: (0, 0))
_out_sds = jax.ShapeDtypeStruct((NP, D), jnp.float32)

_tc_s1 = pl.pallas_call(
    _tc_s1_body, grid=(_GRID,),
    in_specs=[_cnt_spec, _row_spec, _w_spec],
    out_specs=_row_spec, out_shape=_out_sds)

_tc_mid = pl.pallas_call(
    _tc_mid_body, grid=(_GRID,),
    in_specs=[_cnt_spec, _agg_spec, _row_spec, _b_spec, _w_spec],
    out_specs=_row_spec, out_shape=_out_sds)

_tc_out = pl.pallas_call(
    _tc_out_body, grid=(_GRID,),
    in_specs=[_cnt_spec, _agg_spec, _row_spec, _b_spec],
    out_specs=_row_spec, out_shape=_out_sds)


def kernel(x, edge_index, W1, b1, W2, b2):
    src = edge_index[0].astype(jnp.int32)
    dst = edge_index[1].astype(jnp.int32)
    npad = EP - E
    pad_i = jnp.arange(npad, dtype=jnp.int32)
    src_p = jnp.concatenate([src, pad_i % N]).reshape(NW, K, C)
    dst_p = jnp.concatenate([dst, N + pad_i % (NP - N)]).reshape(NW, K, C)

    zeros1 = jnp.zeros((NP,), jnp.float32)
    zeros2 = jnp.zeros((NP, D), jnp.float32)
    xp = jnp.concatenate([x, jnp.zeros((NP - N, D), jnp.float32)], axis=0)

    cnt = _sc_counts(dst_p, zeros1)

    s1 = _tc_s1(cnt, xp, W1)
    agg1 = _sc_agg(s1, src_p, dst_p, zeros2)
    s2 = _tc_mid(cnt, agg1, s1, b1.reshape(1, D), W2)
    agg2 = _sc_agg(s2, src_p, dst_p, zeros2)
    out = _tc_out(cnt, agg2, s2, b2.reshape(1, D))
    return out[:N]
